# baseline (speedup 1.0000x reference)
import jax
import jax.numpy as jnp
from jax.experimental import pallas as pl
from jax.experimental.pallas import tpu as pltpu

_REP = 128
_NSEM = 8


def _body(pe_ref, o_hbm, scratch, sems):
    scratch[...] = jnp.broadcast_to(pe_ref[...], scratch.shape)
    nchunks = o_hbm.shape[0] // _REP
    for j in range(nchunks):
        pltpu.make_async_copy(
            scratch, o_hbm.at[pl.ds(j * _REP, _REP), :], sems.at[j % _NSEM]
        ).start()
    for j in range(nchunks):
        pltpu.make_async_copy(
            scratch, o_hbm.at[pl.ds(j * _REP, _REP), :], sems.at[j % _NSEM]
        ).wait()


def kernel(x, pos_embed):
    batch = x.shape[0]
    max_len, d_model = pos_embed.shape
    row = max_len * d_model
    pe_flat = pos_embed.reshape(1, row)
    out = pl.pallas_call(
        _body,
        in_specs=[pl.BlockSpec((1, row), lambda: (0, 0))],
        out_specs=pl.BlockSpec(memory_space=pltpu.MemorySpace.HBM),
        out_shape=jax.ShapeDtypeStruct((batch, row), jnp.float32),
        scratch_shapes=[
            pltpu.VMEM((_REP, row), jnp.float32),
            pltpu.SemaphoreType.DMA((_NSEM,)),
        ],
    )(pe_flat)
    return out.reshape(batch, max_len, d_model)
